# baseline (speedup 1.0000x reference)
import jax
import jax.numpy as jnp
from jax import lax
from jax.experimental import pallas as pl
from jax.experimental.pallas import tpu as pltpu

_N_LAYERS = 20
_TOKENS = 4096
_F = 1024
_APPLY_TILE = 512
_N_APPLY = _TOKENS // _APPLY_TILE
_CW = 8

_NT = (((1,), (1,)), ((), ()))
_NN = (((1,), (0,)), ((), ()))


def _body(x_ref, w_ref, b_ref, out_ref, m_scr, q_scr, c_scr, crow_scr):
    i = pl.program_id(0)

    @pl.when(i == 0)
    def _init():
        m_scr[...] = w_ref[0]
        c_scr[...] = jnp.broadcast_to(b_ref[0], (_F, _CW))

    @pl.when((i > 0) & (i < _N_LAYERS))
    def _chain():
        w = w_ref[0]
        m_scr[...] = lax.dot_general(
            w, m_scr[...], _NN, preferred_element_type=jnp.float32)
        c_scr[...] = lax.dot_general(
            w, c_scr[...], _NN, preferred_element_type=jnp.float32) + b_ref[0]

    @pl.when(i == _N_LAYERS - 1)
    def _bias_row():
        crow_scr[...] = c_scr[...].T
        q_scr[...] = m_scr[...].T

    @pl.when(i >= _N_LAYERS)
    def _apply():
        out_ref[...] = lax.dot_general(
            x_ref[...], q_scr[...], _NN,
            preferred_element_type=jnp.float32) + crow_scr[0:1, :]


def kernel(x, W, b):
    return pl.pallas_call(
        _body,
        grid=(_N_LAYERS + _N_APPLY,),
        in_specs=[
            pl.BlockSpec((_APPLY_TILE, _F),
                         lambda i: (jnp.maximum(i - _N_LAYERS, 0), 0)),
            pl.BlockSpec((1, _F, _F),
                         lambda i: (jnp.minimum(i, _N_LAYERS - 1), 0, 0)),
            pl.BlockSpec((1, _F, 1),
                         lambda i: (jnp.minimum(i, _N_LAYERS - 1), 0, 0)),
        ],
        out_specs=pl.BlockSpec((_APPLY_TILE, _F),
                               lambda i: (jnp.maximum(i - _N_LAYERS, 0), 0)),
        out_shape=jax.ShapeDtypeStruct((_TOKENS, _F), jnp.float32),
        scratch_shapes=[
            pltpu.VMEM((_F, _F), jnp.float32),
            pltpu.VMEM((_F, _F), jnp.float32),
            pltpu.VMEM((_F, _CW), jnp.float32),
            pltpu.VMEM((_CW, _F), jnp.float32),
        ],
    )(x, W, b.reshape(_N_LAYERS, _F, 1))


# speedup vs baseline: 1.2147x; 1.2147x over previous
import jax
import jax.numpy as jnp
from jax import lax
from jax.experimental import pallas as pl
from jax.experimental.pallas import tpu as pltpu

_N_LAYERS = 20
_TOKENS = 4096
_F = 1024
_AF = _F + 8
_APPLY_TILE = 512
_N_APPLY = _TOKENS // _APPLY_TILE

_NT = (((1,), (1,)), ((), ()))
_NN = (((1,), (0,)), ((), ()))


def _body(x_ref, w_ref, b_ref, out_ref, a_scr):
    i = pl.program_id(0)

    @pl.when(i == 0)
    def _init():
        a_scr[0:_F, :] = w_ref[0].T
        a_scr[_F:, :] = jnp.broadcast_to(b_ref[0], (_AF - _F, _F))

    @pl.when((i > 0) & (i < _N_LAYERS))
    def _chain():
        a_new = lax.dot_general(
            a_scr[...], w_ref[0], _NT, preferred_element_type=jnp.float32)
        a_scr[0:_F, :] = a_new[0:_F, :]
        a_scr[_F:, :] = a_new[_F:, :] + b_ref[0]

    @pl.when(i >= _N_LAYERS)
    def _apply():
        out_ref[...] = lax.dot_general(
            x_ref[...], a_scr[0:_F, :], _NN,
            preferred_element_type=jnp.float32) + a_scr[_F:_F + 1, :]


def kernel(x, W, b):
    return pl.pallas_call(
        _body,
        grid=(_N_LAYERS + _N_APPLY,),
        in_specs=[
            pl.BlockSpec((_APPLY_TILE, _F),
                         lambda i: (jnp.maximum(i - _N_LAYERS, 0), 0)),
            pl.BlockSpec((1, _F, _F),
                         lambda i: (jnp.minimum(i, _N_LAYERS - 1), 0, 0)),
            pl.BlockSpec((1, 1, _F),
                         lambda i: (jnp.minimum(i, _N_LAYERS - 1), 0, 0)),
        ],
        out_specs=pl.BlockSpec((_APPLY_TILE, _F),
                               lambda i: (jnp.maximum(i - _N_LAYERS, 0), 0)),
        out_shape=jax.ShapeDtypeStruct((_TOKENS, _F), jnp.float32),
        scratch_shapes=[
            pltpu.VMEM((_AF, _F), jnp.float32),
        ],
    )(x, W, b.reshape(_N_LAYERS, 1, _F))


# speedup vs baseline: 1.2249x; 1.0084x over previous
import jax
import jax.numpy as jnp
from jax import lax
from jax.experimental import pallas as pl
from jax.experimental.pallas import tpu as pltpu

_N_LAYERS = 20
_TOKENS = 4096
_F = 1024
_AF = _F + 8
_APPLY_TILE = 512
_N_APPLY = _TOKENS // _APPLY_TILE

_NT = (((1,), (1,)), ((), ()))
_NN = (((1,), (0,)), ((), ()))


def _body(x_ref, w_ref, b_ref, out_ref, a_scr):
    i = pl.program_id(0)

    @pl.when(i == 0)
    def _init():
        a_scr[0:_F, :] = w_ref[0].T
        a_scr[_F:, :] = jnp.broadcast_to(b_ref[0], (_AF - _F, _F))

    @pl.when((i > 0) & (i < _N_LAYERS))
    def _chain():
        a_new = lax.dot_general(
            a_scr[...].astype(jnp.bfloat16),
            w_ref[0].astype(jnp.bfloat16),
            _NT, preferred_element_type=jnp.float32)
        a_scr[0:_F, :] = a_new[0:_F, :]
        a_scr[_F:, :] = a_new[_F:, :] + b_ref[0]

    @pl.when(i >= _N_LAYERS)
    def _apply():
        out_ref[...] = lax.dot_general(
            x_ref[...].astype(jnp.bfloat16),
            a_scr[0:_F, :].astype(jnp.bfloat16),
            _NN, preferred_element_type=jnp.float32) + a_scr[_F:_F + 1, :]


def kernel(x, W, b):
    return pl.pallas_call(
        _body,
        grid=(_N_LAYERS + _N_APPLY,),
        in_specs=[
            pl.BlockSpec((_APPLY_TILE, _F),
                         lambda i: (jnp.maximum(i - _N_LAYERS, 0), 0)),
            pl.BlockSpec((1, _F, _F),
                         lambda i: (jnp.minimum(i, _N_LAYERS - 1), 0, 0)),
            pl.BlockSpec((1, 1, _F),
                         lambda i: (jnp.minimum(i, _N_LAYERS - 1), 0, 0)),
        ],
        out_specs=pl.BlockSpec((_APPLY_TILE, _F),
                               lambda i: (jnp.maximum(i - _N_LAYERS, 0), 0)),
        out_shape=jax.ShapeDtypeStruct((_TOKENS, _F), jnp.float32),
        scratch_shapes=[
            pltpu.VMEM((_AF, _F), jnp.float32),
        ],
    )(x, W, b.reshape(_N_LAYERS, 1, _F))


# speedup vs baseline: 1.2321x; 1.0059x over previous
import jax
import jax.numpy as jnp
from jax import lax
from jax.experimental import pallas as pl
from jax.experimental.pallas import tpu as pltpu

_N_LAYERS = 20
_HALF = _N_LAYERS // 2
_TOKENS = 4096
_F = 1024
_AF = _F + 8
_APPLY_TILE = 512
_N_APPLY = _TOKENS // _APPLY_TILE
_MERGE = _HALF
_APPLY0 = _MERGE + 1

_NT = (((1,), (1,)), ((), ()))
_NN = (((1,), (0,)), ((), ()))


def _bf(v):
    return v.astype(jnp.bfloat16)


def _body(x_ref, wa_ref, wb_ref, ba_ref, bb_ref, out_ref, a_scr, b_scr):
    i = pl.program_id(0)

    @pl.when(i == 0)
    def _init():
        a_scr[0:_F, :] = wa_ref[0].T
        a_scr[_F:, :] = jnp.broadcast_to(ba_ref[0], (_AF - _F, _F))
        b_scr[0:_F, :] = wb_ref[0].T
        b_scr[_F:, :] = jnp.broadcast_to(bb_ref[0], (_AF - _F, _F))

    @pl.when((i > 0) & (i < _HALF))
    def _chain():
        a_new = lax.dot_general(
            _bf(a_scr[...]), _bf(wa_ref[0]), _NT,
            preferred_element_type=jnp.float32)
        b_new = lax.dot_general(
            _bf(b_scr[...]), _bf(wb_ref[0]), _NT,
            preferred_element_type=jnp.float32)
        a_scr[0:_F, :] = a_new[0:_F, :]
        a_scr[_F:, :] = a_new[_F:, :] + ba_ref[0]
        b_scr[0:_F, :] = b_new[0:_F, :]
        b_scr[_F:, :] = b_new[_F:, :] + bb_ref[0]

    @pl.when(i == _MERGE)
    def _merge():
        m_new = lax.dot_general(
            _bf(a_scr[...]), _bf(b_scr[0:_F, :]), _NN,
            preferred_element_type=jnp.float32)
        a_scr[0:_F, :] = m_new[0:_F, :]
        a_scr[_F:, :] = m_new[_F:, :] + b_scr[_F:, :]

    @pl.when(i >= _APPLY0)
    def _apply():
        out_ref[...] = lax.dot_general(
            _bf(x_ref[...]), _bf(a_scr[0:_F, :]), _NN,
            preferred_element_type=jnp.float32) + a_scr[_F:_F + 1, :]


def kernel(x, W, b):
    b3 = b.reshape(_N_LAYERS, 1, _F)
    return pl.pallas_call(
        _body,
        grid=(_APPLY0 + _N_APPLY,),
        in_specs=[
            pl.BlockSpec((_APPLY_TILE, _F),
                         lambda i: (jnp.maximum(i - _APPLY0, 0), 0)),
            pl.BlockSpec((1, _F, _F),
                         lambda i: (jnp.minimum(i, _HALF - 1), 0, 0)),
            pl.BlockSpec((1, _F, _F),
                         lambda i: (_HALF + jnp.minimum(i, _HALF - 1), 0, 0)),
            pl.BlockSpec((1, 1, _F),
                         lambda i: (jnp.minimum(i, _HALF - 1), 0, 0)),
            pl.BlockSpec((1, 1, _F),
                         lambda i: (_HALF + jnp.minimum(i, _HALF - 1), 0, 0)),
        ],
        out_specs=pl.BlockSpec((_APPLY_TILE, _F),
                               lambda i: (jnp.maximum(i - _APPLY0, 0), 0)),
        out_shape=jax.ShapeDtypeStruct((_TOKENS, _F), jnp.float32),
        scratch_shapes=[
            pltpu.VMEM((_AF, _F), jnp.float32),
            pltpu.VMEM((_AF, _F), jnp.float32),
        ],
    )(x, W, W, b3, b3)


# speedup vs baseline: 1.2725x; 1.0328x over previous
import jax
import jax.numpy as jnp
from jax import lax
from jax.experimental import pallas as pl
from jax.experimental.pallas import tpu as pltpu

_N_LAYERS = 20
_LPS = 2
_N_CHAIN = _N_LAYERS // _LPS
_TOKENS = 4096
_F = 1024
_AF = _F + 8
_APPLY_TILE = 512
_N_APPLY = _TOKENS // _APPLY_TILE

_NT = (((1,), (1,)), ((), ()))
_NN = (((1,), (0,)), ((), ()))


def _bf(v):
    return v.astype(jnp.bfloat16)


def _fold(a, w, brow):
    t = lax.dot_general(_bf(a), _bf(w), _NT,
                        preferred_element_type=jnp.float32)
    return jnp.concatenate([t[0:_F, :], t[_F:, :] + brow], axis=0)


def _body(x_ref, we_ref, wo_ref, be_ref, bo_ref, out_ref, a_scr):
    i = pl.program_id(0)

    @pl.when(i == 0)
    def _init():
        a0 = jnp.concatenate(
            [we_ref[0].T, jnp.broadcast_to(be_ref[0], (_AF - _F, _F))], axis=0)
        a_scr[...] = _fold(a0, wo_ref[0], bo_ref[0])

    @pl.when((i > 0) & (i < _N_CHAIN))
    def _chain():
        t = _fold(a_scr[...], we_ref[0], be_ref[0])
        a_scr[...] = _fold(t, wo_ref[0], bo_ref[0])

    @pl.when(i >= _N_CHAIN)
    def _apply():
        out_ref[...] = lax.dot_general(
            _bf(x_ref[...]), _bf(a_scr[0:_F, :]), _NN,
            preferred_element_type=jnp.float32) + a_scr[_F:_F + 1, :]


def kernel(x, W, b):
    b3 = b.reshape(_N_LAYERS, 1, _F)
    return pl.pallas_call(
        _body,
        grid=(_N_CHAIN + _N_APPLY,),
        in_specs=[
            pl.BlockSpec((_APPLY_TILE, _F),
                         lambda i: (jnp.maximum(i - _N_CHAIN, 0), 0)),
            pl.BlockSpec((1, _F, _F),
                         lambda i: (2 * jnp.minimum(i, _N_CHAIN - 1), 0, 0)),
            pl.BlockSpec((1, _F, _F),
                         lambda i: (2 * jnp.minimum(i, _N_CHAIN - 1) + 1, 0, 0)),
            pl.BlockSpec((1, 1, _F),
                         lambda i: (2 * jnp.minimum(i, _N_CHAIN - 1), 0, 0)),
            pl.BlockSpec((1, 1, _F),
                         lambda i: (2 * jnp.minimum(i, _N_CHAIN - 1) + 1, 0, 0)),
        ],
        out_specs=pl.BlockSpec((_APPLY_TILE, _F),
                               lambda i: (jnp.maximum(i - _N_CHAIN, 0), 0)),
        out_shape=jax.ShapeDtypeStruct((_TOKENS, _F), jnp.float32),
        scratch_shapes=[
            pltpu.VMEM((_AF, _F), jnp.float32),
        ],
    )(x, W, W, b3, b3)


# speedup vs baseline: 1.2946x; 1.0174x over previous
import jax
import jax.numpy as jnp
from jax import lax
from jax.experimental import pallas as pl
from jax.experimental.pallas import tpu as pltpu

_N_LAYERS = 20
_LPS = 4
_N_CHAIN = _N_LAYERS // _LPS
_TOKENS = 4096
_F = 1024
_AF = _F + 8
_APPLY_TILE = 1024
_N_APPLY = _TOKENS // _APPLY_TILE

_NT = (((1,), (1,)), ((), ()))
_NN = (((1,), (0,)), ((), ()))


def _bf(v):
    return v.astype(jnp.bfloat16)


def _fold(a, w, brow):
    t = lax.dot_general(_bf(a), _bf(w), _NT,
                        preferred_element_type=jnp.float32)
    return jnp.concatenate([t[0:_F, :], t[_F:, :] + brow], axis=0)


def _body(x_ref, w0_ref, w1_ref, w2_ref, w3_ref,
          b0_ref, b1_ref, b2_ref, b3_ref, out_ref, a_scr):
    i = pl.program_id(0)

    @pl.when(i == 0)
    def _init():
        a = jnp.concatenate(
            [w0_ref[0].T, jnp.broadcast_to(b0_ref[0], (_AF - _F, _F))],
            axis=0)
        a = _fold(a, w1_ref[0], b1_ref[0])
        a = _fold(a, w2_ref[0], b2_ref[0])
        a_scr[...] = _fold(a, w3_ref[0], b3_ref[0])

    @pl.when((i > 0) & (i < _N_CHAIN))
    def _chain():
        a = _fold(a_scr[...], w0_ref[0], b0_ref[0])
        a = _fold(a, w1_ref[0], b1_ref[0])
        a = _fold(a, w2_ref[0], b2_ref[0])
        a_scr[...] = _fold(a, w3_ref[0], b3_ref[0])

    @pl.when(i >= _N_CHAIN)
    def _apply():
        out_ref[...] = lax.dot_general(
            _bf(x_ref[...]), _bf(a_scr[0:_F, :]), _NN,
            preferred_element_type=jnp.float32) + a_scr[_F:_F + 1, :]


def kernel(x, W, b):
    b3 = b.reshape(_N_LAYERS, 1, _F)

    def _wspec(j):
        return pl.BlockSpec(
            (1, _F, _F),
            lambda i, j=j: (_LPS * jnp.minimum(i, _N_CHAIN - 1) + j, 0, 0))

    def _bspec(j):
        return pl.BlockSpec(
            (1, 1, _F),
            lambda i, j=j: (_LPS * jnp.minimum(i, _N_CHAIN - 1) + j, 0, 0))

    return pl.pallas_call(
        _body,
        grid=(_N_CHAIN + _N_APPLY,),
        in_specs=[
            pl.BlockSpec((_APPLY_TILE, _F),
                         lambda i: (jnp.maximum(i - _N_CHAIN, 0), 0)),
            _wspec(0), _wspec(1), _wspec(2), _wspec(3),
            _bspec(0), _bspec(1), _bspec(2), _bspec(3),
        ],
        out_specs=pl.BlockSpec((_APPLY_TILE, _F),
                               lambda i: (jnp.maximum(i - _N_CHAIN, 0), 0)),
        out_shape=jax.ShapeDtypeStruct((_TOKENS, _F), jnp.float32),
        scratch_shapes=[
            pltpu.VMEM((_AF, _F), jnp.float32),
        ],
    )(x, W, W, W, W, b3, b3, b3, b3)
